# baseline (speedup 1.0000x reference)
import functools
import math

import jax
import jax.numpy as jnp
from jax import lax
from jax.experimental import pallas as pl
from jax.experimental.pallas import tpu as pltpu
from jax.experimental.pallas import tpu_sc as plsc

N = 10000
B = 8
L = 1250
N1 = 625
E = 320000
D = 128
NL = 3

TW = 144
CH = 40
NWORK = 32
NCHUNK = E // CH
PER_W = NCHUNK // NWORK
ROWCH = N // CH


def _edge_body(ts_hbm, td_hbm, src_hbm, dst_hbm, wd_hbm, wx_hbm, out_hbm,
               tsvA, tdvA, tsvB, tdvB, orow, sidxA, didxA, sidxB, didxB,
               wdv, wxv, acc, semA, semB):
    c = lax.axis_index("c")
    s = lax.axis_index("s")
    wid = s * 2 + c

    pltpu.sync_copy(wd_hbm, wdv)
    pltpu.sync_copy(wx_hbm, wxv)

    zero16 = jnp.zeros((16,), jnp.float32)

    def zrow(e, carry):
        for j in range(TW // 16):
            orow[e, pl.ds(j * 16, 16)] = zero16
        return carry
    lax.fori_loop(0, CH, zrow, 0)

    def zacc(t, carry):
        cc = wid + NWORK * t
        @pl.when(cc < ROWCH)
        def _():
            pltpu.sync_copy(orow, acc.at[pl.ds(cc * CH, CH)])
        return carry
    lax.fori_loop(0, (ROWCH + NWORK - 1) // NWORK, zacc, 0)
    plsc.subcore_barrier()

    lanes = lax.iota(jnp.int32, 16)
    deg1 = jnp.where(lanes == 3, jnp.full((16,), 1.0, jnp.float32),
                     jnp.full((16,), 0.0, jnp.float32))
    magic = jnp.full((16,), 0x5f3759df, jnp.int32)

    def issue(cidx, sidxX, didxX, tsvX, tdvX, semX):
        base = (wid + NWORK * cidx) * CH
        pltpu.sync_copy(src_hbm.at[pl.ds(base, CH)], sidxX)
        pltpu.sync_copy(dst_hbm.at[pl.ds(base, CH)], didxX)
        pltpu.async_copy(ts_hbm.at[sidxX], tsvX, semX)
        pltpu.async_copy(td_hbm.at[didxX], tdvX, semX)

    def wait(sidxX, didxX, tsvX, tdvX, semX):
        pltpu.make_async_copy(ts_hbm.at[sidxX], tsvX, semX).wait()
        pltpu.make_async_copy(td_hbm.at[didxX], tdvX, semX).wait()

    def compute(tsvX, tdvX, didxX):
        def edge(e, ecarry):
            rel = tsvX[e, pl.ds(128, 16)] - tdvX[e, pl.ds(128, 16)]
            r2 = rel * rel
            d2 = r2[0] + r2[1] + r2[2] + 1e-8
            d2v = lax.broadcast(d2, (16,))
            yi = magic - lax.shift_right_arithmetic(
                lax.bitcast_convert_type(d2v, jnp.int32),
                jnp.full((16,), 1, jnp.int32))
            y = lax.bitcast_convert_type(yi, jnp.float32)
            half = d2v * 0.5
            y = y * (1.5 - half * y * y)
            y = y * (1.5 - half * y * y)
            y = y * (1.5 - half * y * y)
            distv = d2v * y
            vsum = jnp.zeros((16,), jnp.float32)
            for j in range(8):
                sl = pl.ds(j * 16, 16)
                mj = jnp.maximum(tsvX[e, sl] + tdvX[e, sl] + distv * wdv[sl],
                                 0.0)
                orow[e, sl] = mj
                vsum = vsum + mj * wxv[sl]
            vals = [vsum[i] for i in range(16)]
            while len(vals) > 1:
                vals = [vals[i] + vals[i + 1] for i in range(0, len(vals), 2)]
            x = jnp.minimum(jnp.maximum(vals[0], -30.0), 30.0)
            ev = jnp.exp(lax.broadcast(2.0 * x, (16,)))
            tv = 1.0 - 2.0 / (ev + 1.0)
            orow[e, pl.ds(128, 16)] = rel * tv + deg1
            return ecarry
        lax.fori_loop(0, CH, edge, 0, unroll=2)
        pltpu.sync_copy(orow, acc.at[didxX], add=True)

    issue(0, sidxA, didxA, tsvA, tdvA, semA)

    def pair(g, carry):
        wait(sidxA, didxA, tsvA, tdvA, semA)
        issue(2 * g + 1, sidxB, didxB, tsvB, tdvB, semB)
        compute(tsvA, tdvA, didxA)
        wait(sidxB, didxB, tsvB, tdvB, semB)
        issue(2 * g + 2, sidxA, didxA, tsvA, tdvA, semA)
        compute(tsvB, tdvB, didxB)
        return carry
    lax.fori_loop(0, (PER_W - 2) // 2, pair, 0)
    wait(sidxA, didxA, tsvA, tdvA, semA)
    issue(PER_W - 1, sidxB, didxB, tsvB, tdvB, semB)
    compute(tsvA, tdvA, didxA)
    wait(sidxB, didxB, tsvB, tdvB, semB)
    compute(tsvB, tdvB, didxB)
    plsc.subcore_barrier()

    def dump(t, carry):
        cc = s + 16 * t
        @pl.when(cc < ROWCH)
        def _():
            pltpu.sync_copy(acc.at[pl.ds(cc * CH, CH)],
                            out_hbm.at[c, pl.ds(cc * CH, CH)])
        return carry
    lax.fori_loop(0, (ROWCH + 15) // 16, dump, 0)


_edge_call = functools.partial(
    pl.kernel,
    out_type=jax.ShapeDtypeStruct((2, N, TW), jnp.float32),
    mesh=plsc.VectorSubcoreMesh(core_axis_name="c", subcore_axis_name="s"),
    compiler_params=pltpu.CompilerParams(use_tc_tiling_on_sc=False),
    scratch_types=[
        pltpu.VMEM((CH, TW), jnp.float32),
        pltpu.VMEM((CH, TW), jnp.float32),
        pltpu.VMEM((CH, TW), jnp.float32),
        pltpu.VMEM((CH, TW), jnp.float32),
        pltpu.VMEM((CH, TW), jnp.float32),
        pltpu.VMEM((CH,), jnp.int32),
        pltpu.VMEM((CH,), jnp.int32),
        pltpu.VMEM((CH,), jnp.int32),
        pltpu.VMEM((CH,), jnp.int32),
        pltpu.VMEM((D,), jnp.float32),
        pltpu.VMEM((D,), jnp.float32),
        pltpu.VMEM_SHARED((N, TW), jnp.float32),
        pltpu.SemaphoreType.DMA,
        pltpu.SemaphoreType.DMA,
    ],
)(_edge_body)


def _edge_stage(TS, TD, src, dst, wd, wx):
    out = _edge_call(TS, TD, src, dst, wd, wx)
    return out[0] + out[1]


def _mm_body(act, x_ref, w_ref, b_ref, o_ref):
    acc = jnp.dot(x_ref[...], w_ref[...], preferred_element_type=jnp.float32)
    acc = acc + b_ref[0:1, :]
    if act == "relu":
        acc = jnp.maximum(acc, 0.0)
    o_ref[...] = acc


def _mm(x, w, b, act=None, block_r=400):
    r, k = x.shape
    dout = w.shape[1]
    b2 = jnp.broadcast_to(b.reshape(1, dout), (8, dout))
    grid = (r // block_r,)
    return pl.pallas_call(
        functools.partial(_mm_body, act),
        grid=grid,
        in_specs=[
            pl.BlockSpec((block_r, k), lambda i: (i, 0)),
            pl.BlockSpec((k, dout), lambda i: (0, 0)),
            pl.BlockSpec((8, dout), lambda i: (0, 0)),
        ],
        out_specs=pl.BlockSpec((block_r, dout), lambda i: (i, 0)),
        out_shape=jax.ShapeDtypeStruct((r, dout), jnp.float32),
    )(x, w, b2)


def _att_body(h_ref, o_ref, wq_ref, wk_ref, wv_ref, out_ref):
    q = jnp.dot(h_ref[0], wq_ref[...], preferred_element_type=jnp.float32)
    kk = jnp.dot(o_ref[0], wk_ref[...], preferred_element_type=jnp.float32)
    vv = jnp.dot(o_ref[0], wv_ref[...], preferred_element_type=jnp.float32)
    s = lax.dot_general(q, kk, (((1,), (1,)), ((), ())),
                        preferred_element_type=jnp.float32)
    s = s * (1.0 / math.sqrt(32.0))
    mx = jnp.max(s, axis=1, keepdims=True)
    p = jnp.exp(s - mx)
    att = p / jnp.sum(p, axis=1, keepdims=True)
    out_ref[0] = jnp.dot(att, vv, preferred_element_type=jnp.float32)


def _attention(c3, o3, wq, wk, wv, block_r=L):
    grid = (B, L // block_r)
    out = pl.pallas_call(
        _att_body,
        grid=grid,
        in_specs=[
            pl.BlockSpec((1, block_r, D), lambda b, r: (b, r, 0)),
            pl.BlockSpec((1, L, D), lambda b, r: (b, 0, 0)),
            pl.BlockSpec((D, 32), lambda b, r: (0, 0)),
            pl.BlockSpec((D, 32), lambda b, r: (0, 0)),
            pl.BlockSpec((D, D), lambda b, r: (0, 0)),
        ],
        out_specs=pl.BlockSpec((1, block_r, D), lambda b, r: (b, r, 0)),
        out_shape=jax.ShapeDtypeStruct((B, L, D), jnp.float32),
    )(c3, o3, wq, wk, wv)
    return out.reshape(N, D)


def _pool_body(c_ref, w1_ref, b1_ref, w2_ref, b2_ref, out_ref):
    rows = lax.broadcasted_iota(jnp.int32, (L, D), 0)
    mask = jnp.where(rows < N1, 1.0, 0.0)
    pooled = jnp.sum(c_ref[0] * mask, axis=0, keepdims=True) * (1.0 / N1)
    x = jnp.maximum(
        jnp.dot(pooled, w1_ref[...], preferred_element_type=jnp.float32)
        + b1_ref[0:1, :], 0.0)
    y = jnp.dot(x, w2_ref[...], preferred_element_type=jnp.float32) + b2_ref[0:1, :]
    y = 1.0 / (1.0 + jnp.exp(-y))
    out_ref[0] = jnp.broadcast_to(y, (8, D))


def _pool_head(c3, w1, b1, w2, b2):
    w2p = jnp.concatenate([w2, jnp.zeros((D, D - 1), jnp.float32)], axis=1)
    b1b = jnp.broadcast_to(b1.reshape(1, D), (8, D))
    b2b = jnp.broadcast_to(
        jnp.concatenate([b2, jnp.zeros((D - 1,), jnp.float32)]).reshape(1, D),
        (8, D))
    out = pl.pallas_call(
        _pool_body,
        grid=(B,),
        in_specs=[
            pl.BlockSpec((1, L, D), lambda b: (b, 0, 0)),
            pl.BlockSpec((D, D), lambda b: (0, 0)),
            pl.BlockSpec((8, D), lambda b: (0, 0)),
            pl.BlockSpec((D, D), lambda b: (0, 0)),
            pl.BlockSpec((8, D), lambda b: (0, 0)),
        ],
        out_specs=pl.BlockSpec((1, 8, D), lambda b: (b, 0, 0)),
        out_shape=jax.ShapeDtypeStruct((B, 8, D), jnp.float32),
    )(c3, w1, b1b, w2p, b2b)
    return out[:, 0, 0]


def kernel(coords, feat, edge_index, cross_edge_index, c_valid, W_embede,
           W_msg, b_msg, W_x, Wq, Wk, Wv, W_h, b_h, FC_W1, FC_b1, FC_W2,
           FC_b2):
    del c_valid
    c_hs = _mm(feat, W_embede, jnp.zeros((D,), jnp.float32))
    orig3 = c_hs.reshape(B, L, D)
    X = coords
    src_a, dst_a = edge_index[0], edge_index[1]
    src_b, dst_b = cross_edge_index[0], cross_edge_index[1]
    zpad = jnp.zeros((N, TW - D - 3), jnp.float32)

    for k in range(NL):
        Wm = W_msg[k]
        HS = _mm(c_hs, Wm[:D], b_msg[k])
        HD = _mm(c_hs, Wm[D:2 * D], jnp.zeros((D,), jnp.float32))
        wd = Wm[2 * D]
        wx = W_x[k][:, 0]
        att_out = _attention(c_hs.reshape(B, L, D), orig3, Wq[k], Wk[k], Wv[k])
        cs = []
        for src, dst in ((src_a, dst_a), (src_b, dst_b)):
            Xp = jnp.concatenate([X, zpad], axis=1)
            TS = jnp.concatenate([HS, Xp], axis=1)
            TD = jnp.concatenate([HD, Xp], axis=1)
            acc = _edge_stage(TS, TD, src, dst, wd, wx)
            agg = acc[:, :D]
            cagg = acc[:, D:D + 3]
            deg = acc[:, D + 3:D + 4] + 1.0
            X = X + cagg / deg
            h_new = _mm(jnp.concatenate([c_hs, agg, att_out], axis=1),
                        W_h[k], b_h[k], act="relu")
            cs.append(h_new)
        c_hs = cs[1] - cs[0]

    return _pool_head(c_hs.reshape(B, L, D), FC_W1, FC_b1, FC_W2, FC_b2)


# speedup vs baseline: 1.1389x; 1.1389x over previous
import functools
import math

import jax
import jax.numpy as jnp
from jax import lax
from jax.experimental import pallas as pl
from jax.experimental.pallas import tpu as pltpu
from jax.experimental.pallas import tpu_sc as plsc

N = 10000
B = 8
L = 1250
N1 = 625
E = 320000
D = 128
NL = 3

TW = 144
CH = 80
NWORK = 32
NCHUNK = E // CH
PER_W = NCHUNK // NWORK
ROWCH = N // CH


def _edge_body(ts_hbm, td_hbm, src_hbm, dst_hbm, wd_hbm, wx_hbm, out_hbm,
               tsvA, tdvA, orow, sidxA, didxA, wdv, wxv, acc, semA, semB):
    c = lax.axis_index("c")
    s = lax.axis_index("s")
    wid = s * 2 + c

    pltpu.sync_copy(wd_hbm, wdv)
    pltpu.sync_copy(wx_hbm, wxv)

    zero16 = jnp.zeros((16,), jnp.float32)

    def zrow(e, carry):
        for j in range(TW // 16):
            orow[e, pl.ds(j * 16, 16)] = zero16
        return carry
    lax.fori_loop(0, CH, zrow, 0)

    def zacc(t, carry):
        cc = wid + NWORK * t
        @pl.when(cc < ROWCH)
        def _():
            pltpu.sync_copy(orow, acc.at[pl.ds(cc * CH, CH)])
        return carry
    lax.fori_loop(0, (ROWCH + NWORK - 1) // NWORK, zacc, 0)
    plsc.subcore_barrier()

    lanes = lax.iota(jnp.int32, 16)
    deg1 = jnp.where(lanes == 3, jnp.full((16,), 1.0, jnp.float32),
                     jnp.full((16,), 0.0, jnp.float32))
    magic = jnp.full((16,), 0x5f3759df, jnp.int32)

    def compute(tsvX, tdvX, didxX):
        def edge(e, ecarry):
            rel = tsvX[e, pl.ds(128, 16)] - tdvX[e, pl.ds(128, 16)]
            r2 = rel * rel
            d2 = r2[0] + r2[1] + r2[2] + 1e-8
            d2v = lax.broadcast(d2, (16,))
            yi = magic - lax.shift_right_arithmetic(
                lax.bitcast_convert_type(d2v, jnp.int32),
                jnp.full((16,), 1, jnp.int32))
            y = lax.bitcast_convert_type(yi, jnp.float32)
            half = d2v * 0.5
            y = y * (1.5 - half * y * y)
            y = y * (1.5 - half * y * y)
            y = y * (1.5 - half * y * y)
            distv = d2v * y
            vsum = jnp.zeros((16,), jnp.float32)
            for j in range(8):
                sl = pl.ds(j * 16, 16)
                mj = jnp.maximum(tsvX[e, sl] + tdvX[e, sl] + distv * wdv[sl],
                                 0.0)
                orow[e, sl] = mj
                vsum = vsum + mj * wxv[sl]
            vals = [vsum[i] for i in range(16)]
            while len(vals) > 1:
                vals = [vals[i] + vals[i + 1] for i in range(0, len(vals), 2)]
            x = jnp.minimum(jnp.maximum(vals[0], -30.0), 30.0)
            ev = jnp.exp(lax.broadcast(2.0 * x, (16,)))
            tv = 1.0 - 2.0 / (ev + 1.0)
            orow[e, pl.ds(128, 16)] = rel * tv + deg1
            return ecarry
        lax.fori_loop(0, CH, edge, 0)
        pltpu.sync_copy(orow, acc.at[didxX], add=True)

    def chunk(t, carry):
        base = (wid + NWORK * t) * CH
        pltpu.sync_copy(src_hbm.at[pl.ds(base, CH)], sidxA)
        pltpu.sync_copy(dst_hbm.at[pl.ds(base, CH)], didxA)
        pltpu.async_copy(ts_hbm.at[sidxA], tsvA, semA).wait()
        pltpu.async_copy(td_hbm.at[didxA], tdvA, semB).wait()
        compute(tsvA, tdvA, didxA)
        return carry
    lax.fori_loop(0, PER_W, chunk, 0)
    plsc.subcore_barrier()

    def dump(t, carry):
        cc = s + 16 * t
        @pl.when(cc < ROWCH)
        def _():
            pltpu.sync_copy(acc.at[pl.ds(cc * CH, CH)],
                            out_hbm.at[c, pl.ds(cc * CH, CH)])
        return carry
    lax.fori_loop(0, (ROWCH + 15) // 16, dump, 0)


_edge_call = functools.partial(
    pl.kernel,
    out_type=jax.ShapeDtypeStruct((2, N, TW), jnp.float32),
    mesh=plsc.VectorSubcoreMesh(core_axis_name="c", subcore_axis_name="s"),
    compiler_params=pltpu.CompilerParams(use_tc_tiling_on_sc=False),
    scratch_types=[
        pltpu.VMEM((CH, TW), jnp.float32),
        pltpu.VMEM((CH, TW), jnp.float32),
        pltpu.VMEM((CH, TW), jnp.float32),
        pltpu.VMEM((CH,), jnp.int32),
        pltpu.VMEM((CH,), jnp.int32),
        pltpu.VMEM((D,), jnp.float32),
        pltpu.VMEM((D,), jnp.float32),
        pltpu.VMEM_SHARED((N, TW), jnp.float32),
        pltpu.SemaphoreType.DMA,
        pltpu.SemaphoreType.DMA,
    ],
)(_edge_body)


def _edge_stage(TS, TD, src, dst, wd, wx):
    out = _edge_call(TS, TD, src, dst, wd, wx)
    return out[0] + out[1]


def _mm_body(act, x_ref, w_ref, b_ref, o_ref):
    acc = jnp.dot(x_ref[...], w_ref[...], preferred_element_type=jnp.float32)
    acc = acc + b_ref[0:1, :]
    if act == "relu":
        acc = jnp.maximum(acc, 0.0)
    o_ref[...] = acc


def _mm(x, w, b, act=None, block_r=400):
    r, k = x.shape
    dout = w.shape[1]
    b2 = jnp.broadcast_to(b.reshape(1, dout), (8, dout))
    grid = (r // block_r,)
    return pl.pallas_call(
        functools.partial(_mm_body, act),
        grid=grid,
        in_specs=[
            pl.BlockSpec((block_r, k), lambda i: (i, 0)),
            pl.BlockSpec((k, dout), lambda i: (0, 0)),
            pl.BlockSpec((8, dout), lambda i: (0, 0)),
        ],
        out_specs=pl.BlockSpec((block_r, dout), lambda i: (i, 0)),
        out_shape=jax.ShapeDtypeStruct((r, dout), jnp.float32),
    )(x, w, b2)


def _att_body(h_ref, o_ref, wq_ref, wk_ref, wv_ref, out_ref):
    q = jnp.dot(h_ref[0], wq_ref[...], preferred_element_type=jnp.float32)
    kk = jnp.dot(o_ref[0], wk_ref[...], preferred_element_type=jnp.float32)
    vv = jnp.dot(o_ref[0], wv_ref[...], preferred_element_type=jnp.float32)
    s = lax.dot_general(q, kk, (((1,), (1,)), ((), ())),
                        preferred_element_type=jnp.float32)
    s = s * (1.0 / math.sqrt(32.0))
    mx = jnp.max(s, axis=1, keepdims=True)
    p = jnp.exp(s - mx)
    att = p / jnp.sum(p, axis=1, keepdims=True)
    out_ref[0] = jnp.dot(att, vv, preferred_element_type=jnp.float32)


def _attention(c3, o3, wq, wk, wv, block_r=L):
    grid = (B, L // block_r)
    out = pl.pallas_call(
        _att_body,
        grid=grid,
        in_specs=[
            pl.BlockSpec((1, block_r, D), lambda b, r: (b, r, 0)),
            pl.BlockSpec((1, L, D), lambda b, r: (b, 0, 0)),
            pl.BlockSpec((D, 32), lambda b, r: (0, 0)),
            pl.BlockSpec((D, 32), lambda b, r: (0, 0)),
            pl.BlockSpec((D, D), lambda b, r: (0, 0)),
        ],
        out_specs=pl.BlockSpec((1, block_r, D), lambda b, r: (b, r, 0)),
        out_shape=jax.ShapeDtypeStruct((B, L, D), jnp.float32),
    )(c3, o3, wq, wk, wv)
    return out.reshape(N, D)


def _pool_body(c_ref, w1_ref, b1_ref, w2_ref, b2_ref, out_ref):
    rows = lax.broadcasted_iota(jnp.int32, (L, D), 0)
    mask = jnp.where(rows < N1, 1.0, 0.0)
    pooled = jnp.sum(c_ref[0] * mask, axis=0, keepdims=True) * (1.0 / N1)
    x = jnp.maximum(
        jnp.dot(pooled, w1_ref[...], preferred_element_type=jnp.float32)
        + b1_ref[0:1, :], 0.0)
    y = jnp.dot(x, w2_ref[...], preferred_element_type=jnp.float32) + b2_ref[0:1, :]
    y = 1.0 / (1.0 + jnp.exp(-y))
    out_ref[0] = jnp.broadcast_to(y, (8, D))


def _pool_head(c3, w1, b1, w2, b2):
    w2p = jnp.concatenate([w2, jnp.zeros((D, D - 1), jnp.float32)], axis=1)
    b1b = jnp.broadcast_to(b1.reshape(1, D), (8, D))
    b2b = jnp.broadcast_to(
        jnp.concatenate([b2, jnp.zeros((D - 1,), jnp.float32)]).reshape(1, D),
        (8, D))
    out = pl.pallas_call(
        _pool_body,
        grid=(B,),
        in_specs=[
            pl.BlockSpec((1, L, D), lambda b: (b, 0, 0)),
            pl.BlockSpec((D, D), lambda b: (0, 0)),
            pl.BlockSpec((8, D), lambda b: (0, 0)),
            pl.BlockSpec((D, D), lambda b: (0, 0)),
            pl.BlockSpec((8, D), lambda b: (0, 0)),
        ],
        out_specs=pl.BlockSpec((1, 8, D), lambda b: (b, 0, 0)),
        out_shape=jax.ShapeDtypeStruct((B, 8, D), jnp.float32),
    )(c3, w1, b1b, w2p, b2b)
    return out[:, 0, 0]


def kernel(coords, feat, edge_index, cross_edge_index, c_valid, W_embede,
           W_msg, b_msg, W_x, Wq, Wk, Wv, W_h, b_h, FC_W1, FC_b1, FC_W2,
           FC_b2):
    del c_valid
    c_hs = _mm(feat, W_embede, jnp.zeros((D,), jnp.float32))
    orig3 = c_hs.reshape(B, L, D)
    X = coords
    src_a, dst_a = edge_index[0], edge_index[1]
    src_b, dst_b = cross_edge_index[0], cross_edge_index[1]
    zpad = jnp.zeros((N, TW - D - 3), jnp.float32)

    for k in range(NL):
        Wm = W_msg[k]
        HS = _mm(c_hs, Wm[:D], b_msg[k])
        HD = _mm(c_hs, Wm[D:2 * D], jnp.zeros((D,), jnp.float32))
        wd = Wm[2 * D]
        wx = W_x[k][:, 0]
        att_out = _attention(c_hs.reshape(B, L, D), orig3, Wq[k], Wk[k], Wv[k])
        cs = []
        for src, dst in ((src_a, dst_a), (src_b, dst_b)):
            Xp = jnp.concatenate([X, zpad], axis=1)
            TS = jnp.concatenate([HS, Xp], axis=1)
            TD = jnp.concatenate([HD, Xp], axis=1)
            acc = _edge_stage(TS, TD, src, dst, wd, wx)
            agg = acc[:, :D]
            cagg = acc[:, D:D + 3]
            deg = acc[:, D + 3:D + 4] + 1.0
            X = X + cagg / deg
            h_new = _mm(jnp.concatenate([c_hs, agg, att_out], axis=1),
                        W_h[k], b_h[k], act="relu")
            cs.append(h_new)
        c_hs = cs[1] - cs[0]

    return _pool_head(c_hs.reshape(B, L, D), FC_W1, FC_b1, FC_W2, FC_b2)
